# baseline (speedup 1.0000x reference)
import functools
import jax
import jax.numpy as jnp
from jax import lax
from jax.experimental import pallas as pl
from jax.experimental.pallas import tpu as pltpu
from jax.experimental.pallas import tpu_sc as plsc

B, L, D, V = 4, 2048, 768, 30522
NC, NS, LANES = 2, 16, 16
NW = NC * NS
C = L // NW
BR = 512


def _gather_body(x_hbm, word_hbm, out_hbm, idx_v, buf_v, sem):
    wid = lax.axis_index("s") * NC + lax.axis_index("c")
    l0 = wid * C
    pltpu.sync_copy(x_hbm.at[pl.ds(l0, C)], idx_v)
    pltpu.async_copy(word_hbm.at[idx_v], buf_v, sem).wait()
    pltpu.sync_copy(buf_v, out_hbm.at[pl.ds(l0, C)])


_mesh = plsc.VectorSubcoreMesh(core_axis_name="c", subcore_axis_name="s",
                               num_cores=NC, num_subcores=NS)

_sc_gather_row = functools.partial(
    pl.kernel,
    out_type=jax.ShapeDtypeStruct((L, D), jnp.float32),
    mesh=_mesh,
    scratch_types=[
        pltpu.VMEM((C,), jnp.int32),
        pltpu.VMEM((C, D), jnp.float32),
        pltpu.SemaphoreType.DMA,
    ],
)(_gather_body)


def _ln_compute(g_ref, seg_ref, pos_ref, tok_ref, gamma_ref, beta_ref, o_ref):
    segf = seg_ref[0, 0].astype(jnp.float32)[:, None]
    tok0 = tok_ref[0, :]
    tokd = tok_ref[1, :] - tok0
    emb = (g_ref[...] + pos_ref[...]
           + (tok0[None, :] + segf * tokd[None, :]))
    mean = jnp.mean(emb, axis=-1, keepdims=True)
    cent = emb - mean
    var = jnp.mean(cent * cent, axis=-1, keepdims=True)
    rstd = lax.rsqrt(var + 1e-12)
    o_ref[0] = (cent * rstd * gamma_ref[...][None, :]
                + beta_ref[...][None, :])


def _ln_body_first(g_ref, seg_ref, pos_ref, tok_ref, gamma_ref, beta_ref,
                   o_ref):
    _ln_compute(g_ref, seg_ref, pos_ref, tok_ref, gamma_ref, beta_ref, o_ref)


def _ln_body_alias(g_ref, seg_ref, pos_ref, tok_ref, gamma_ref, beta_ref,
                   prev_ref, o_ref):
    _ln_compute(g_ref, seg_ref, pos_ref, tok_ref, gamma_ref, beta_ref, o_ref)


def _make_tc_ln(b, alias):
    in_specs = [
        pl.BlockSpec((BR, D), lambda i: (i, 0)),
        pl.BlockSpec((1, 1, BR), lambda i, _b=b: (_b, 0, i)),
        pl.BlockSpec((BR, D), lambda i: (i, 0)),
        pl.BlockSpec((8, D), lambda i: (0, 0)),
        pl.BlockSpec((D,), lambda i: (0,)),
        pl.BlockSpec((D,), lambda i: (0,)),
    ]
    kwargs = {}
    body = _ln_body_first
    if alias:
        in_specs.append(pl.BlockSpec(memory_space=pltpu.HBM))
        kwargs["input_output_aliases"] = {6: 0}
        body = _ln_body_alias
    return pl.pallas_call(
        body,
        grid=(L // BR,),
        in_specs=in_specs,
        out_specs=pl.BlockSpec((1, BR, D), lambda i, _b=b: (_b, i, 0)),
        out_shape=jax.ShapeDtypeStruct((B, L, D), jnp.float32),
        **kwargs,
    )


_tc_ln = [_make_tc_ln(b, alias=(b > 0)) for b in range(B)]


@jax.jit
def kernel(x, segs, word_emb, pos_emb, tok_emb, gamma, beta):
    xi = x.astype(jnp.int32)
    si = segs.astype(jnp.int32).reshape(B, 1, L)
    g = [_sc_gather_row(xi[b], word_emb) for b in range(B)]
    out = _tc_ln[0](g[0], si, pos_emb, tok_emb, gamma, beta)
    for b in range(1, B):
        out = _tc_ln[b](g[b], si, pos_emb, tok_emb, gamma, beta, out)
    return out


# speedup vs baseline: 1.1850x; 1.1850x over previous
import functools
import jax
import jax.numpy as jnp
from jax import lax
from jax.experimental import pallas as pl
from jax.experimental.pallas import tpu as pltpu
from jax.experimental.pallas import tpu_sc as plsc

B, L, D, V = 4, 2048, 768, 30522
NC, NS, LANES = 2, 16, 16
NW = NC * NS
C = L // NW
N = B * L

BR = 512


def _gather_body(x_hbm, word_hbm, out_hbm,
                 idx0, idx1, buf0, buf1, gs0, gs1, ws0, ws1):
    wid = lax.axis_index("s") * NC + lax.axis_index("c")
    l0 = wid * C
    idx = (idx0, idx1)
    buf = (buf0, buf1)
    gsem = (gs0, gs1)
    wsem = (ws0, ws1)

    pltpu.sync_copy(x_hbm.at[0, pl.ds(l0, C)], idx0)
    gathers = [pltpu.async_copy(word_hbm.at[idx0], buf0, gs0)]
    writes = [None, None]
    for b in range(B):
        p = b % 2
        q = (b + 1) % 2
        if b + 1 < B:
            pltpu.sync_copy(x_hbm.at[b + 1, pl.ds(l0, C)], idx[q])
            if writes[q] is not None:
                writes[q].wait()
                writes[q] = None
            gathers.append(pltpu.async_copy(word_hbm.at[idx[q]], buf[q],
                                            gsem[q]))
        gathers[b].wait()
        writes[p] = pltpu.async_copy(buf[p],
                                     out_hbm.at[b, pl.ds(l0, C)], wsem[p])
    for w in writes:
        if w is not None:
            w.wait()


_mesh = plsc.VectorSubcoreMesh(core_axis_name="c", subcore_axis_name="s",
                               num_cores=NC, num_subcores=NS)

_sc_gather = functools.partial(
    pl.kernel,
    out_type=jax.ShapeDtypeStruct((B, L, D), jnp.float32),
    mesh=_mesh,
    scratch_types=[
        pltpu.VMEM((C,), jnp.int32),
        pltpu.VMEM((C,), jnp.int32),
        pltpu.VMEM((C, D), jnp.float32),
        pltpu.VMEM((C, D), jnp.float32),
        pltpu.SemaphoreType.DMA,
        pltpu.SemaphoreType.DMA,
        pltpu.SemaphoreType.DMA,
        pltpu.SemaphoreType.DMA,
    ],
)(_gather_body)


def _ln_body(g_ref, seg_ref, pos_ref, tok_ref, gamma_ref, beta_ref, o_ref):
    segf = seg_ref[...].astype(jnp.float32)[..., None]
    tok0 = tok_ref[0, :]
    tokd = tok_ref[1, :] - tok0
    emb = (g_ref[...] + pos_ref[...][None]
           + (tok0[None, None, :] + segf * tokd[None, None, :]))
    mean = jnp.mean(emb, axis=-1, keepdims=True)
    cent = emb - mean
    var = jnp.mean(cent * cent, axis=-1, keepdims=True)
    rstd = lax.rsqrt(var + 1e-12)
    o_ref[...] = (cent * rstd * gamma_ref[...][None, None, :]
                  + beta_ref[...][None, None, :])


_tc_ln = pl.pallas_call(
    _ln_body,
    grid=(L // BR,),
    in_specs=[
        pl.BlockSpec((B, BR, D), lambda i: (0, i, 0)),
        pl.BlockSpec((B, BR), lambda i: (0, i)),
        pl.BlockSpec((BR, D), lambda i: (i, 0)),
        pl.BlockSpec((8, D), lambda i: (0, 0)),
        pl.BlockSpec((D,), lambda i: (0,)),
        pl.BlockSpec((D,), lambda i: (0,)),
    ],
    out_specs=pl.BlockSpec((B, BR, D), lambda i: (0, i, 0)),
    out_shape=jax.ShapeDtypeStruct((B, L, D), jnp.float32),
)


@jax.jit
def kernel(x, segs, word_emb, pos_emb, tok_emb, gamma, beta):
    gathered = _sc_gather(x.astype(jnp.int32), word_emb)
    return _tc_ln(gathered, segs.astype(jnp.int32), pos_emb, tok_emb,
                  gamma, beta)
